# Initial kernel scaffold; baseline (speedup 1.0000x reference)
import jax
import jax.numpy as jnp
from jax.experimental import pallas as pl


def kernel(metadata_ids, emb_weight):
    raise NotImplementedError("write your pallas kernel here")


# speedup vs baseline: 1.1005x; 1.1005x over previous
import functools

import jax
import jax.numpy as jnp
from jax import lax
from jax.experimental import pallas as pl
from jax.experimental.pallas import tpu as pltpu
from jax.experimental.pallas import tpu_sc as plsc

_VOCAB = 100000
_DIM = 64
_NC = 2
_NS = 16
_NW = _NC * _NS
_CHUNK = 128


def _make_gather(n_rows: int):
    assert n_rows % (_NW * _CHUNK) == 0
    n_chunks = n_rows // (_NW * _CHUNK)
    mesh = plsc.VectorSubcoreMesh(core_axis_name="c", subcore_axis_name="s")

    @functools.partial(
        pl.kernel,
        out_type=jax.ShapeDtypeStruct((_NW, n_chunks, _CHUNK, _DIM), jnp.float32),
        mesh=mesh,
        scratch_types=[
            pltpu.VMEM((n_chunks, _CHUNK), jnp.int32),
            pltpu.VMEM((_CHUNK, _DIM), jnp.float32),
            pltpu.SemaphoreType.DMA,
        ],
        compiler_params=pltpu.CompilerParams(use_tc_tiling_on_sc=False),
    )
    def gather_kernel(table_hbm, idx_hbm, out_hbm, idx_v, buf, sem):
        wid = lax.axis_index("s") * _NC + lax.axis_index("c")
        pltpu.sync_copy(idx_hbm.at[wid], idx_v)

        def body(j, carry):
            pltpu.async_copy(table_hbm.at[idx_v.at[j]], buf, sem).wait()
            pltpu.sync_copy(buf, out_hbm.at[wid, j])
            return carry

        lax.fori_loop(0, n_chunks, body, 0, unroll=False)

    return gather_kernel


def kernel(metadata_ids, emb_weight):
    batch, n_fields = metadata_ids.shape
    n_rows = batch * n_fields
    n_chunks = n_rows // (_NW * _CHUNK)
    idx = metadata_ids.astype(jnp.int32).reshape(_NW, n_chunks, _CHUNK)
    out = _make_gather(n_rows)(emb_weight, idx)
    return out.reshape(batch, n_fields, _DIM)


# speedup vs baseline: 1.2140x; 1.1031x over previous
import functools

import jax
import jax.numpy as jnp
from jax import lax
from jax.experimental import pallas as pl
from jax.experimental.pallas import tpu as pltpu
from jax.experimental.pallas import tpu_sc as plsc

_VOCAB = 100000
_DIM = 64
_NC = 2
_NS = 16
_NW = _NC * _NS
_CHUNK = 128


_NBUF = 4


def _make_gather(n_rows: int):
    assert n_rows % (_NW * _CHUNK) == 0
    n_chunks = n_rows // (_NW * _CHUNK)
    n_outer = (n_chunks + _NBUF - 1) // _NBUF
    mesh = plsc.VectorSubcoreMesh(core_axis_name="c", subcore_axis_name="s")

    @functools.partial(
        pl.kernel,
        out_type=jax.ShapeDtypeStruct((_NW, n_chunks, _CHUNK, _DIM), jnp.float32),
        mesh=mesh,
        scratch_types=(
            [pltpu.VMEM((n_chunks, _CHUNK), jnp.int32)]
            + [pltpu.VMEM((_CHUNK, _DIM), jnp.float32) for _ in range(_NBUF)]
            + [pltpu.SemaphoreType.DMA for _ in range(2 * _NBUF)]
        ),
        compiler_params=pltpu.CompilerParams(use_tc_tiling_on_sc=False),
    )
    def gather_kernel(table_hbm, idx_hbm, out_hbm, idx_v, *scratch):
        bufs = scratch[:_NBUF]
        gsems = scratch[_NBUF:2 * _NBUF]
        wsems = scratch[2 * _NBUF:]
        wid = lax.axis_index("s") * _NC + lax.axis_index("c")
        pltpu.sync_copy(idx_hbm.at[wid], idx_v)

        def start_gather(j, b):
            pltpu.async_copy(table_hbm.at[idx_v.at[j]], bufs[b], gsems[b])

        for b in range(_NBUF - 1):
            start_gather(b, b)

        def outer(t, carry):
            for b in range(_NBUF):
                j = t * _NBUF + b

                @pl.when(j < n_chunks)
                def _():
                    pltpu.make_async_copy(
                        table_hbm.at[idx_v.at[j]], bufs[b], gsems[b]
                    ).wait()
                    pltpu.async_copy(bufs[b], out_hbm.at[wid, j], wsems[b])

                jn = j + _NBUF - 1
                bn = (b + _NBUF - 1) % _NBUF

                @pl.when(jn < n_chunks)
                def _():
                    @pl.when(jn >= _NBUF)
                    def _():
                        pltpu.make_async_copy(
                            bufs[bn], out_hbm.at[wid, jn], wsems[bn]
                        ).wait()

                    start_gather(jn, bn)

            return carry

        lax.fori_loop(0, n_outer, outer, 0, unroll=False)

        for b in range(_NBUF):
            j_last = n_chunks - _NBUF + (b - n_chunks) % _NBUF
            pltpu.make_async_copy(
                bufs[b], out_hbm.at[wid, j_last], wsems[b]
            ).wait()

    return gather_kernel


def kernel(metadata_ids, emb_weight):
    batch, n_fields = metadata_ids.shape
    n_rows = batch * n_fields
    n_chunks = n_rows // (_NW * _CHUNK)
    idx = metadata_ids.astype(jnp.int32).reshape(_NW, n_chunks, _CHUNK)
    out = _make_gather(n_rows)(emb_weight, idx)
    return out.reshape(batch, n_fields, _DIM)
